# baseline (speedup 1.0000x reference)
import functools
import math

import jax
import jax.numpy as jnp
from jax import lax
from jax.experimental import pallas as pl
from jax.experimental.pallas import tpu as pltpu
from jax.experimental.pallas import tpu_sc as plsc

_EMBED = 64
_SCALE = math.sqrt(_EMBED)
_LANES = 16
_NW = 32
_BLK = 128
_FH = _EMBED // 8
_NBUF = 4


def _make_embed(n_s, n_bh):
    n_blocks = n_s * n_bh
    assert n_blocks % (_NW * _NBUF) == 0 and n_blocks // _NW >= 2 * _NBUF
    bpw = n_blocks // _NW
    mesh = plsc.VectorSubcoreMesh(core_axis_name="c", subcore_axis_name="s")

    scratch = (
        [pltpu.VMEM((bpw, _BLK), jnp.int32)]
        + [pltpu.VMEM((_BLK, _EMBED), jnp.float32) for _ in range(_NBUF)]
        + [pltpu.VMEM((_FH, 8 * _BLK), jnp.float32) for _ in range(_NBUF)]
        + [pltpu.SemaphoreType.DMA for _ in range(2 * _NBUF)]
    )

    @functools.partial(
        pl.kernel,
        mesh=mesh,
        out_type=jax.ShapeDtypeStruct((n_s * _FH * n_bh, 8 * _BLK), jnp.float32),
        scratch_types=scratch,
        compiler_params=pltpu.CompilerParams(
            use_tc_tiling_on_sc=False, needs_layout_passes=False),
    )
    def embed(idx_hbm, table_hbm, out_hbm, idx_v, *refs):
        bufs = refs[:_NBUF]
        obufs = refs[_NBUF:2 * _NBUF]
        sem_g = refs[2 * _NBUF:3 * _NBUF]
        sem_s = refs[3 * _NBUF:]
        wid = lax.axis_index("s") * 2 + lax.axis_index("c")
        g0 = wid * bpw
        iota = lax.iota(jnp.int32, _LANES)

        pltpu.sync_copy(idx_hbm.at[pl.ds(g0, bpw)], idx_v)

        def gather_start(i, b):
            pltpu.async_copy(table_hbm.at[idx_v.at[i]], bufs[b], sem_g[b])

        def gather_wait(b):
            pltpu.make_async_copy(
                table_hbm.at[idx_v.at[0]], bufs[b], sem_g[b]).wait()

        def transpose_scale(b):
            @plsc.parallel_loop(0, _FH * 8 * (_BLK // _LANES), step=1, unroll=8)
            def _(t):
                fh = t // (8 * (_BLK // _LANES))
                r = t % (8 * (_BLK // _LANES))
                fl = r // (_BLK // _LANES)
                bk = r % (_BLK // _LANES)
                rows = bk * _LANES + iota
                cols = iota * 0 + (fh * 8 + fl)
                v = plsc.load_gather(bufs[b], [rows, cols])
                obufs[b][fh, pl.ds(fl * _BLK + bk * _LANES, _LANES)] = v * _SCALE

        def scatter_start(i, b):
            g = g0 + i
            s = g // n_bh
            bh = g % n_bh
            for fh in range(_FH):
                pltpu.async_copy(
                    obufs[b].at[fh],
                    out_hbm.at[(s * _FH + fh) * n_bh + bh],
                    sem_s[b])

        def scatter_wait(b):
            pltpu.make_async_copy(
                obufs[b], out_hbm.at[pl.ds(0, _FH)], sem_s[b]).wait()

        gather_start(0, 0)
        gather_start(1, 1)

        def outer(oi, carry):
            c0 = oi * _NBUF
            for j in range(_NBUF):
                c = c0 + j
                b = j
                nb = (j + 2) % _NBUF
                gather_wait(b)

                @pl.when(c >= 2)
                def _():
                    scatter_wait(nb)

                @pl.when(c + 2 < bpw)
                def _():
                    gather_start(c + 2, nb)

                transpose_scale(b)
                scatter_start(c, b)
            return carry

        lax.fori_loop(0, bpw // _NBUF, outer, 0)

        scatter_wait((bpw - 2) % _NBUF)
        scatter_wait((bpw - 1) % _NBUF)

    return embed


def kernel(input_token, table):
    batch, seq = input_token.shape
    n_bh = batch // _BLK
    idx2 = jnp.swapaxes(input_token, 0, 1).reshape(seq * n_bh, _BLK)
    idx2 = idx2.astype(jnp.int32)
    out2 = _make_embed(seq, n_bh)(idx2, table)
    out5 = out2.reshape(seq, _FH, n_bh, 8, _BLK)
    return out5.transpose(2, 4, 0, 1, 3).reshape(batch, seq, _EMBED)


# speedup vs baseline: 1.2274x; 1.2274x over previous
import functools
import math

import jax
import jax.numpy as jnp
from jax import lax
from jax.experimental import pallas as pl
from jax.experimental.pallas import tpu as pltpu
from jax.experimental.pallas import tpu_sc as plsc

_EMBED = 64
_SCALE = math.sqrt(_EMBED)
_LANES = 16
_NC = 2
_NS = 16
_NW = _NC * _NS

_CHUNK = 256
_NBUF = 4


def _make_embed(total):
    assert total % (_NW * _CHUNK) == 0
    b_per_w = total // _NW
    n_chunks = b_per_w // _CHUNK
    assert n_chunks % _NBUF == 0 and n_chunks >= 2 * _NBUF
    mesh = plsc.VectorSubcoreMesh(core_axis_name="c", subcore_axis_name="s")

    scratch = (
        [pltpu.VMEM((b_per_w,), jnp.int32)]
        + [pltpu.VMEM((_CHUNK, _EMBED), jnp.float32) for _ in range(_NBUF)]
        + [pltpu.SemaphoreType.DMA for _ in range(2 * _NBUF)]
    )

    @functools.partial(
        pl.kernel,
        mesh=mesh,
        out_type=jax.ShapeDtypeStruct((total, _EMBED), jnp.float32),
        scratch_types=scratch,
        compiler_params=pltpu.CompilerParams(use_tc_tiling_on_sc=False),
    )
    def embed(idx_hbm, table_hbm, out_hbm, idx_v, *bufs_and_sems):
        rows = bufs_and_sems[:_NBUF]
        sem_g = bufs_and_sems[_NBUF:2 * _NBUF]
        sem_s = bufs_and_sems[2 * _NBUF:]
        wid = lax.axis_index("s") * _NC + lax.axis_index("c")
        base = wid * b_per_w

        pltpu.sync_copy(idx_hbm.at[pl.ds(base, b_per_w)], idx_v)

        def gather_start(c, b):
            pltpu.async_copy(
                table_hbm.at[idx_v.at[pl.ds(c * _CHUNK, _CHUNK)]],
                rows[b], sem_g[b])

        def gather_wait(b):
            pltpu.make_async_copy(
                table_hbm.at[idx_v.at[pl.ds(0, _CHUNK)]],
                rows[b], sem_g[b]).wait()

        def scatter_start(c, b):
            pltpu.async_copy(
                rows[b], out_hbm.at[pl.ds(base + c * _CHUNK, _CHUNK)],
                sem_s[b])

        def scatter_wait(b):
            pltpu.make_async_copy(
                rows[b], out_hbm.at[pl.ds(base, _CHUNK)], sem_s[b]).wait()

        def scale(b):
            @plsc.parallel_loop(0, _CHUNK, step=1, unroll=8)
            def _(i):
                for j in range(_EMBED // _LANES):
                    sl = pl.ds(j * _LANES, _LANES)
                    rows[b][i, sl] = rows[b][i, sl] * _SCALE

        gather_start(0, 0)
        gather_start(1, 1)

        def outer(oi, carry):
            c0 = oi * _NBUF
            for j in range(_NBUF):
                c = c0 + j
                b = j
                nb = (j + 2) % _NBUF
                gather_wait(b)

                @pl.when(c >= 2)
                def _():
                    scatter_wait(nb)

                @pl.when(c + 2 < n_chunks)
                def _():
                    gather_start(c + 2, nb)

                scale(b)
                scatter_start(c, b)
            return carry

        lax.fori_loop(0, n_chunks // _NBUF, outer, 0)

        scatter_wait((n_chunks - 2) % _NBUF)
        scatter_wait((n_chunks - 1) % _NBUF)

    return embed


def kernel(input_token, table):
    batch, seq = input_token.shape
    total = batch * seq
    idx = input_token.reshape(total).astype(jnp.int32)
    out = _make_embed(total)(idx, table)
    return out.reshape(batch, seq, _EMBED)


# speedup vs baseline: 1.2276x; 1.0002x over previous
import functools
import math

import jax
import jax.numpy as jnp
from jax import lax
from jax.experimental import pallas as pl
from jax.experimental.pallas import tpu as pltpu
from jax.experimental.pallas import tpu_sc as plsc

_EMBED = 64
_SCALE = math.sqrt(_EMBED)
_LANES = 16
_NC = 2
_NS = 16
_NW = _NC * _NS

_CHUNK = 128
_NBUF = 8
_LOOK = 4


def _make_embed(total):
    assert total % (_NW * _CHUNK) == 0
    b_per_w = total // _NW
    n_chunks = b_per_w // _CHUNK
    assert n_chunks % _NBUF == 0 and n_chunks >= 2 * _NBUF
    mesh = plsc.VectorSubcoreMesh(core_axis_name="c", subcore_axis_name="s")

    scratch = (
        [pltpu.VMEM((b_per_w,), jnp.int32)]
        + [pltpu.VMEM((_CHUNK, _EMBED), jnp.float32) for _ in range(_NBUF)]
        + [pltpu.SemaphoreType.DMA for _ in range(2 * _NBUF)]
    )

    @functools.partial(
        pl.kernel,
        mesh=mesh,
        out_type=jax.ShapeDtypeStruct((total, _EMBED), jnp.float32),
        scratch_types=scratch,
        compiler_params=pltpu.CompilerParams(use_tc_tiling_on_sc=False),
    )
    def embed(idx_hbm, table_hbm, out_hbm, idx_v, *bufs_and_sems):
        rows = bufs_and_sems[:_NBUF]
        sem_g = bufs_and_sems[_NBUF:2 * _NBUF]
        sem_s = bufs_and_sems[2 * _NBUF:]
        wid = lax.axis_index("s") * _NC + lax.axis_index("c")
        base = wid * b_per_w

        pltpu.sync_copy(idx_hbm.at[pl.ds(base, b_per_w)], idx_v)

        def gather_start(c, b):
            pltpu.async_copy(
                table_hbm.at[idx_v.at[pl.ds(c * _CHUNK, _CHUNK)]],
                rows[b], sem_g[b])

        def gather_wait(b):
            pltpu.make_async_copy(
                table_hbm.at[idx_v.at[pl.ds(0, _CHUNK)]],
                rows[b], sem_g[b]).wait()

        def scatter_start(c, b):
            pltpu.async_copy(
                rows[b], out_hbm.at[pl.ds(base + c * _CHUNK, _CHUNK)],
                sem_s[b])

        def scatter_wait(b):
            pltpu.make_async_copy(
                rows[b], out_hbm.at[pl.ds(base, _CHUNK)], sem_s[b]).wait()

        def scale(b):
            @plsc.parallel_loop(0, _CHUNK, step=1, unroll=8)
            def _(i):
                for j in range(_EMBED // _LANES):
                    sl = pl.ds(j * _LANES, _LANES)
                    rows[b][i, sl] = rows[b][i, sl] * _SCALE

        for c in range(_LOOK):
            gather_start(c, c)

        def outer(oi, carry):
            c0 = oi * _NBUF
            for j in range(_NBUF):
                c = c0 + j
                b = j
                nb = (j + _LOOK) % _NBUF
                gather_wait(b)

                @pl.when(c + _LOOK >= _NBUF)
                def _():
                    scatter_wait(nb)

                @pl.when(c + _LOOK < n_chunks)
                def _():
                    gather_start(c + _LOOK, nb)

                scale(b)
                scatter_start(c, b)
            return carry

        lax.fori_loop(0, n_chunks // _NBUF, outer, 0)

        for k in range(_NBUF - _LOOK, _NBUF):
            scatter_wait((n_chunks - _NBUF + k) % _NBUF)

    return embed


def kernel(input_token, table):
    batch, seq = input_token.shape
    total = batch * seq
    idx = input_token.reshape(total).astype(jnp.int32)
    out = _make_embed(total)(idx, table)
    return out.reshape(batch, seq, _EMBED)
